# baseline (speedup 1.0000x reference)
import jax
import jax.numpy as jnp
from jax import lax
from jax.experimental import pallas as pl
from jax.experimental.pallas import tpu as pltpu
from jax.experimental.pallas import tpu_sc as plsc

N = 10000
E = 320000
D = 128
H = 8
C = 16
HC = H * C
NPAD = 10240
ROWW = 144
NC = 2
NS = 16
NW = NC * NS
EPW = E // NW
CH = 128
NCH = 80
PADE = NCH * CH - EPW
BN = 1000
RPT = NPAD // NS

_HIGH = lax.Precision.HIGHEST


def _tc1_body(x_ref, wq_ref, bq_ref, ab_ref, q_ref, qa_ref, gm_ref):
    xq = jnp.dot(x_ref[...], wq_ref[...], precision=_HIGH) + bq_ref[...]
    q_ref[...] = xq
    qa = jnp.dot(xq, ab_ref[...], precision=_HIGH)
    qa_ref[...] = qa
    bm = jnp.max(qa, axis=0, keepdims=True)
    i = pl.program_id(0)

    @pl.when(i == 0)
    def _():
        gm_ref[...] = bm

    @pl.when(i != 0)
    def _():
        gm_ref[...] = jnp.maximum(gm_ref[...], bm)


def _tc2_body(q_ref, qa_ref, gm_ref, p_ref, e8_ref, t_ref):
    qz = jnp.exp(qa_ref[...] - gm_ref[...])
    qzrep = jnp.dot(qz, p_ref[...], precision=_HIGH)
    t_ref[:, :HC] = q_ref[...] * qzrep
    t_ref[:, HC:ROWW] = jnp.dot(qz, e8_ref[...], precision=_HIGH)


def _tc3_body(a0_ref, a1_ref, p_ref, o_ref):
    s = a0_ref[0] + a1_ref[0]
    den = s[:, HC:HC + H]
    dexp = jnp.dot(den, p_ref[...], precision=_HIGH)
    o_ref[...] = jnp.maximum(s[:, :HC] / (dexp + 1e-16), 0.0)


def _sc_body(src_hbm, dst_hbm, t_hbm, z_hbm, out_hbm,
             sidx, didx, rows, accum, sem):
    c = lax.axis_index("c")
    s = lax.axis_index("s")
    base = pl.multiple_of((c * NS + s) * NCH * CH, 8)
    rb = pl.multiple_of(s * RPT, 8)

    pltpu.sync_copy(z_hbm.at[pl.ds(rb, RPT)], accum.at[pl.ds(rb, RPT)])
    plsc.subcore_barrier()

    def body(j, carry):
        eb = pl.multiple_of(base + j * CH, 8)
        pltpu.sync_copy(src_hbm.at[pl.ds(eb, CH)], sidx)
        pltpu.sync_copy(dst_hbm.at[pl.ds(eb, CH)], didx)
        pltpu.async_copy(t_hbm.at[sidx], rows, sem).wait()
        pltpu.sync_copy(rows, accum.at[didx], add=True)
        return carry

    lax.fori_loop(0, NCH, body, 0, unroll=False)

    plsc.subcore_barrier()
    pltpu.sync_copy(accum.at[pl.ds(rb, RPT)], out_hbm.at[c, pl.ds(rb, RPT)])


def kernel(x, Wq, bq, Wk, bk, A, edge_index):
    del Wk, bk
    f32 = jnp.float32
    ab = (A.T[:, :, None] * jnp.eye(H, dtype=f32)[:, None, :]).reshape(HC, H)
    p_exp = jnp.kron(jnp.eye(H, dtype=f32), jnp.ones((1, C), f32))
    e8 = jnp.concatenate([jnp.eye(H, dtype=f32),
                          jnp.zeros((H, ROWW - HC - H), f32)], axis=1)
    bq2 = bq.reshape(1, HC)
    srcp = jnp.pad(edge_index[0].reshape(NW, EPW), ((0, 0), (0, PADE)),
                   constant_values=0).reshape(NW * NCH * CH)
    dstp = jnp.pad(edge_index[1].reshape(NW, EPW), ((0, 0), (0, PADE)),
                   constant_values=NPAD - 1).reshape(NW * NCH * CH)
    zrows = jnp.zeros((NPAD, ROWW), f32)

    grid = N // BN
    q, qa, gm = pl.pallas_call(
        _tc1_body,
        grid=(grid,),
        in_specs=[
            pl.BlockSpec((BN, D), lambda i: (i, 0)),
            pl.BlockSpec((D, HC), lambda i: (0, 0)),
            pl.BlockSpec((1, HC), lambda i: (0, 0)),
            pl.BlockSpec((HC, H), lambda i: (0, 0)),
        ],
        out_specs=[
            pl.BlockSpec((BN, HC), lambda i: (i, 0)),
            pl.BlockSpec((BN, H), lambda i: (i, 0)),
            pl.BlockSpec((1, H), lambda i: (0, 0)),
        ],
        out_shape=[
            jax.ShapeDtypeStruct((N, HC), f32),
            jax.ShapeDtypeStruct((N, H), f32),
            jax.ShapeDtypeStruct((1, H), f32),
        ],
    )(x, Wq, bq2, ab)

    t_tab = pl.pallas_call(
        _tc2_body,
        grid=(grid,),
        in_specs=[
            pl.BlockSpec((BN, HC), lambda i: (i, 0)),
            pl.BlockSpec((BN, H), lambda i: (i, 0)),
            pl.BlockSpec((1, H), lambda i: (0, 0)),
            pl.BlockSpec((H, HC), lambda i: (0, 0)),
            pl.BlockSpec((H, ROWW - HC), lambda i: (0, 0)),
        ],
        out_specs=pl.BlockSpec((BN, ROWW), lambda i: (i, 0)),
        out_shape=jax.ShapeDtypeStruct((N, ROWW), f32),
    )(q, qa, gm, p_exp, e8)

    mesh = plsc.VectorSubcoreMesh(core_axis_name="c", subcore_axis_name="s",
                                  num_cores=NC, num_subcores=NS)
    acc = pl.kernel(
        _sc_body,
        out_type=jax.ShapeDtypeStruct((NC, NPAD, ROWW), f32),
        mesh=mesh,
        scratch_types=[
            pltpu.VMEM((CH,), jnp.int32),
            pltpu.VMEM((CH,), jnp.int32),
            pltpu.VMEM((CH, ROWW), f32),
            pltpu.VMEM_SHARED((NPAD, ROWW), f32),
            pltpu.SemaphoreType.DMA,
        ],
        compiler_params=pltpu.CompilerParams(use_tc_tiling_on_sc=False),
    )(srcp, dstp, t_tab, zrows)

    out = pl.pallas_call(
        _tc3_body,
        grid=(grid,),
        in_specs=[
            pl.BlockSpec((1, BN, ROWW), lambda i: (0, i, 0)),
            pl.BlockSpec((1, BN, ROWW), lambda i: (1, i, 0)),
            pl.BlockSpec((H, HC), lambda i: (0, 0)),
        ],
        out_specs=pl.BlockSpec((BN, HC), lambda i: (i, 0)),
        out_shape=jax.ShapeDtypeStruct((N, HC), f32),
    )(acc, acc, p_exp)

    return out


# speedup vs baseline: 1.7741x; 1.7741x over previous
import jax
import jax.numpy as jnp
from jax import lax
from jax.experimental import pallas as pl
from jax.experimental.pallas import tpu as pltpu
from jax.experimental.pallas import tpu_sc as plsc

N = 10000
E = 320000
D = 128
H = 8
C = 16
HC = H * C
NPAD = 10240
ROWW = 144
NC = 2
NS = 16
NW = NC * NS
EPW = E // NW
CH = 128
NFULL = EPW // CH
TAIL = EPW - NFULL * CH
BN = 1024
RPT = NPAD // NS

_HIGH = lax.Precision.HIGHEST


def _tc1_body(x_ref, wq_ref, bq_ref, ab_ref, q_ref, qa_ref, gm_ref):
    xq = jnp.dot(x_ref[...], wq_ref[...], precision=_HIGH) + bq_ref[...]
    q_ref[...] = xq
    qa = jnp.dot(xq, ab_ref[...], precision=_HIGH)
    qa_ref[...] = qa
    bm = jnp.max(qa, axis=0, keepdims=True)
    i = pl.program_id(0)

    @pl.when(i == 0)
    def _():
        gm_ref[...] = bm

    @pl.when(i != 0)
    def _():
        gm_ref[...] = jnp.maximum(gm_ref[...], bm)


def _tc2_body(q_ref, qa_ref, gm_ref, p_ref, e8_ref, t_ref):
    qz = jnp.exp(qa_ref[...] - gm_ref[...])
    qzrep = jnp.dot(qz, p_ref[...], precision=_HIGH)
    t_ref[:, :HC] = q_ref[...] * qzrep
    t_ref[:, HC:ROWW] = jnp.dot(qz, e8_ref[...], precision=_HIGH)


def _tc3_body(a0_ref, a1_ref, p_ref, o_ref):
    s = a0_ref[...] + a1_ref[...]
    den = s[:, HC:HC + H]
    dexp = jnp.dot(den, p_ref[...], precision=_HIGH)
    o_ref[...] = jnp.maximum(s[:, :HC] / (dexp + 1e-16), 0.0)


def _sc_body(t_hbm, src_hbm, dst_hbm, z_hbm, out_hbm,
             sidx, didx, rows, sidx_t, didx_t, rows_t, accum, sem):
    c = lax.axis_index("c")
    s = lax.axis_index("s")
    base = pl.multiple_of((c * NS + s) * EPW, 8)
    rb = pl.multiple_of(s * RPT, 8)

    pltpu.sync_copy(z_hbm.at[pl.ds(rb, RPT)], accum.at[pl.ds(rb, RPT)])
    plsc.subcore_barrier()

    def body(j, carry):
        eb = pl.multiple_of(base + j * CH, 8)
        pltpu.sync_copy(src_hbm.at[pl.ds(eb, CH)], sidx)
        pltpu.sync_copy(dst_hbm.at[pl.ds(eb, CH)], didx)
        pltpu.async_copy(t_hbm.at[sidx], rows, sem).wait()
        pltpu.sync_copy(rows, accum.at[didx], add=True)
        return carry

    lax.fori_loop(0, NFULL, body, 0, unroll=False)

    eb = pl.multiple_of(base + NFULL * CH, 8)
    pltpu.sync_copy(src_hbm.at[pl.ds(eb, TAIL)], sidx_t)
    pltpu.sync_copy(dst_hbm.at[pl.ds(eb, TAIL)], didx_t)
    pltpu.async_copy(t_hbm.at[sidx_t], rows_t, sem).wait()
    pltpu.sync_copy(rows_t, accum.at[didx_t], add=True)

    plsc.subcore_barrier()
    pltpu.sync_copy(accum.at[pl.ds(rb, RPT)],
                    out_hbm.at[pl.ds(c * NPAD + rb, RPT)])


def kernel(x, Wq, bq, Wk, bk, A, edge_index):
    del Wk, bk
    f32 = jnp.float32
    x_pad = jnp.pad(x, ((0, NPAD - N), (0, 0)))
    ab = (A.T[:, :, None] * jnp.eye(H, dtype=f32)[:, None, :]).reshape(HC, H)
    p_exp = jnp.kron(jnp.eye(H, dtype=f32), jnp.ones((1, C), f32))
    e8 = jnp.concatenate([jnp.eye(H, dtype=f32),
                          jnp.zeros((H, ROWW - HC - H), f32)], axis=1)
    bq2 = bq.reshape(1, HC)
    src = edge_index[0]
    dst = edge_index[1]
    zrows = jnp.zeros((NPAD, ROWW), f32)

    grid = NPAD // BN
    q, qa, gm = pl.pallas_call(
        _tc1_body,
        grid=(grid,),
        in_specs=[
            pl.BlockSpec((BN, D), lambda i: (i, 0)),
            pl.BlockSpec((D, HC), lambda i: (0, 0)),
            pl.BlockSpec((1, HC), lambda i: (0, 0)),
            pl.BlockSpec((HC, H), lambda i: (0, 0)),
        ],
        out_specs=[
            pl.BlockSpec((BN, HC), lambda i: (i, 0)),
            pl.BlockSpec((BN, H), lambda i: (i, 0)),
            pl.BlockSpec((1, H), lambda i: (0, 0)),
        ],
        out_shape=[
            jax.ShapeDtypeStruct((NPAD, HC), f32),
            jax.ShapeDtypeStruct((NPAD, H), f32),
            jax.ShapeDtypeStruct((1, H), f32),
        ],
    )(x_pad, Wq, bq2, ab)

    t_tab = pl.pallas_call(
        _tc2_body,
        grid=(grid,),
        in_specs=[
            pl.BlockSpec((BN, HC), lambda i: (i, 0)),
            pl.BlockSpec((BN, H), lambda i: (i, 0)),
            pl.BlockSpec((1, H), lambda i: (0, 0)),
            pl.BlockSpec((H, HC), lambda i: (0, 0)),
            pl.BlockSpec((H, ROWW - HC), lambda i: (0, 0)),
        ],
        out_specs=pl.BlockSpec((BN, ROWW), lambda i: (i, 0)),
        out_shape=jax.ShapeDtypeStruct((NPAD, ROWW), f32),
    )(q, qa, gm, p_exp, e8)

    mesh = plsc.VectorSubcoreMesh(core_axis_name="c", subcore_axis_name="s",
                                  num_cores=NC, num_subcores=NS)
    acc = pl.kernel(
        _sc_body,
        out_type=jax.ShapeDtypeStruct((NC * NPAD, ROWW), f32),
        mesh=mesh,
        scratch_types=[
            pltpu.VMEM((CH,), jnp.int32),
            pltpu.VMEM((CH,), jnp.int32),
            pltpu.VMEM((CH, ROWW), f32),
            pltpu.VMEM((TAIL,), jnp.int32),
            pltpu.VMEM((TAIL,), jnp.int32),
            pltpu.VMEM((TAIL, ROWW), f32),
            pltpu.VMEM_SHARED((NPAD, ROWW), f32),
            pltpu.SemaphoreType.DMA,
        ],
        compiler_params=pltpu.CompilerParams(use_tc_tiling_on_sc=False),
    )(t_tab, src, dst, zrows)

    out_full = pl.pallas_call(
        _tc3_body,
        grid=(grid,),
        in_specs=[
            pl.BlockSpec((BN, ROWW), lambda i: (i, 0)),
            pl.BlockSpec((BN, ROWW), lambda i: (i + grid, 0)),
            pl.BlockSpec((H, HC), lambda i: (0, 0)),
        ],
        out_specs=pl.BlockSpec((BN, HC), lambda i: (i, 0)),
        out_shape=jax.ShapeDtypeStruct((NPAD, HC), f32),
    )(acc, acc, p_exp)

    return out_full[:N]


# speedup vs baseline: 2.4114x; 1.3592x over previous
import jax
import jax.numpy as jnp
from jax import lax
from jax.experimental import pallas as pl
from jax.experimental.pallas import tpu as pltpu
from jax.experimental.pallas import tpu_sc as plsc

N = 10000
E = 320000
D = 128
H = 8
C = 16
HC = H * C
NPAD = 10240
ROWW = 144
NC = 2
NS = 16
NW = NC * NS
EPW = E // NW
CH = 128
NFULL = EPW // CH
TAIL = EPW - NFULL * CH
BN = 1024
RPT = NPAD // NS

_HIGH = lax.Precision.HIGHEST


def _tc1_body(x_ref, wq_ref, bq_ref, ab_ref, q_ref, qa_ref, gm_ref):
    xq = jnp.dot(x_ref[...], wq_ref[...], precision=_HIGH) + bq_ref[...]
    q_ref[...] = xq
    qa = jnp.dot(xq, ab_ref[...], precision=_HIGH)
    qa_ref[...] = qa
    bm = jnp.max(qa, axis=0, keepdims=True)
    i = pl.program_id(0)

    @pl.when(i == 0)
    def _():
        gm_ref[...] = bm

    @pl.when(i != 0)
    def _():
        gm_ref[...] = jnp.maximum(gm_ref[...], bm)


def _tc2_body(q_ref, qa_ref, gm_ref, p_ref, e8_ref, t_ref):
    qz = jnp.exp(qa_ref[...] - gm_ref[...])
    qzrep = jnp.dot(qz, p_ref[...], precision=_HIGH)
    t_ref[:, :HC] = q_ref[...] * qzrep
    t_ref[:, HC:ROWW] = jnp.dot(qz, e8_ref[...], precision=_HIGH)


def _tc3_body(a0_ref, a1_ref, p_ref, o_ref):
    s = a0_ref[...] + a1_ref[...]
    den = s[:, HC:HC + H]
    dexp = jnp.dot(den, p_ref[...], precision=_HIGH)
    o_ref[...] = jnp.maximum(s[:, :HC] / (dexp + 1e-16), 0.0)


def _sc_body(t_hbm, src_hbm, dst_hbm, z_hbm, out_hbm,
             sidx0, didx0, sidx1, didx1, rows0, rows1, didx_t, accum,
             sem0, sem1):
    c = lax.axis_index("c")
    s = lax.axis_index("s")
    base = pl.multiple_of((c * NS + s) * EPW, 8)
    rb = pl.multiple_of(s * RPT, 8)

    def ldidx(j, sidx, didx):
        eb = pl.multiple_of(base + j * CH, 8)
        pltpu.sync_copy(src_hbm.at[pl.ds(eb, CH)], sidx)
        pltpu.sync_copy(dst_hbm.at[pl.ds(eb, CH)], didx)

    def start(sidx, buf, sem):
        pltpu.async_copy(t_hbm.at[sidx], buf, sem)

    def wait(buf, sem):
        pltpu.make_async_copy(t_hbm.at[sidx0], buf, sem).wait()

    def scat(didx, buf):
        pltpu.sync_copy(buf, accum.at[didx], add=True)

    pltpu.sync_copy(z_hbm.at[pl.ds(rb, RPT)], accum.at[pl.ds(rb, RPT)])
    ldidx(0, sidx0, didx0)
    start(sidx0, rows0, sem0)
    plsc.subcore_barrier()

    def body(jj, carry):
        j = 2 * jj
        ldidx(j + 1, sidx1, didx1)
        start(sidx1, rows1, sem1)
        wait(rows0, sem0)
        scat(didx0, rows0)
        ldidx(j + 2, sidx0, didx0)
        start(sidx0, rows0, sem0)
        wait(rows1, sem1)
        scat(didx1, rows1)
        return carry

    lax.fori_loop(0, (NFULL - 2) // 2, body, 0, unroll=False)

    ldidx(NFULL - 1, sidx1, didx1)
    start(sidx1, rows1, sem1)
    wait(rows0, sem0)
    scat(didx0, rows0)
    wait(rows1, sem1)
    scat(didx1, rows1)

    eb = pl.multiple_of(base + NFULL * CH, 8)
    pltpu.sync_copy(src_hbm.at[pl.ds(eb, TAIL)], sidx0.at[pl.ds(0, TAIL)])
    pltpu.sync_copy(dst_hbm.at[pl.ds(eb, TAIL)], didx_t)
    pltpu.async_copy(t_hbm.at[sidx0.at[pl.ds(0, TAIL)]],
                     rows0.at[pl.ds(0, TAIL)], sem0).wait()
    pltpu.sync_copy(rows0.at[pl.ds(0, TAIL)], accum.at[didx_t], add=True)

    plsc.subcore_barrier()
    pltpu.sync_copy(accum.at[pl.ds(rb, RPT)],
                    out_hbm.at[pl.ds(c * NPAD + rb, RPT)])


def kernel(x, Wq, bq, Wk, bk, A, edge_index):
    del Wk, bk
    f32 = jnp.float32
    x_pad = jnp.pad(x, ((0, NPAD - N), (0, 0)))
    ab = (A.T[:, :, None] * jnp.eye(H, dtype=f32)[:, None, :]).reshape(HC, H)
    p_exp = jnp.kron(jnp.eye(H, dtype=f32), jnp.ones((1, C), f32))
    e8 = jnp.concatenate([jnp.eye(H, dtype=f32),
                          jnp.zeros((H, ROWW - HC - H), f32)], axis=1)
    bq2 = bq.reshape(1, HC)
    src = edge_index[0]
    dst = edge_index[1]
    zrows = jnp.zeros((NPAD, ROWW), f32)

    grid = NPAD // BN
    q, qa, gm = pl.pallas_call(
        _tc1_body,
        grid=(grid,),
        in_specs=[
            pl.BlockSpec((BN, D), lambda i: (i, 0)),
            pl.BlockSpec((D, HC), lambda i: (0, 0)),
            pl.BlockSpec((1, HC), lambda i: (0, 0)),
            pl.BlockSpec((HC, H), lambda i: (0, 0)),
        ],
        out_specs=[
            pl.BlockSpec((BN, HC), lambda i: (i, 0)),
            pl.BlockSpec((BN, H), lambda i: (i, 0)),
            pl.BlockSpec((1, H), lambda i: (0, 0)),
        ],
        out_shape=[
            jax.ShapeDtypeStruct((NPAD, HC), f32),
            jax.ShapeDtypeStruct((NPAD, H), f32),
            jax.ShapeDtypeStruct((1, H), f32),
        ],
    )(x_pad, Wq, bq2, ab)

    t_tab = pl.pallas_call(
        _tc2_body,
        grid=(grid,),
        in_specs=[
            pl.BlockSpec((BN, HC), lambda i: (i, 0)),
            pl.BlockSpec((BN, H), lambda i: (i, 0)),
            pl.BlockSpec((1, H), lambda i: (0, 0)),
            pl.BlockSpec((H, HC), lambda i: (0, 0)),
            pl.BlockSpec((H, ROWW - HC), lambda i: (0, 0)),
        ],
        out_specs=pl.BlockSpec((BN, ROWW), lambda i: (i, 0)),
        out_shape=jax.ShapeDtypeStruct((NPAD, ROWW), f32),
    )(q, qa, gm, p_exp, e8)

    mesh = plsc.VectorSubcoreMesh(core_axis_name="c", subcore_axis_name="s",
                                  num_cores=NC, num_subcores=NS)
    acc = pl.kernel(
        _sc_body,
        out_type=jax.ShapeDtypeStruct((NC * NPAD, ROWW), f32),
        mesh=mesh,
        scratch_types=[
            pltpu.VMEM((CH,), jnp.int32),
            pltpu.VMEM((CH,), jnp.int32),
            pltpu.VMEM((CH,), jnp.int32),
            pltpu.VMEM((CH,), jnp.int32),
            pltpu.VMEM((CH, ROWW), f32),
            pltpu.VMEM((CH, ROWW), f32),
            pltpu.VMEM((TAIL,), jnp.int32),
            pltpu.VMEM_SHARED((NPAD, ROWW), f32),
            pltpu.SemaphoreType.DMA,
            pltpu.SemaphoreType.DMA,
        ],
        compiler_params=pltpu.CompilerParams(use_tc_tiling_on_sc=False),
    )(t_tab, src, dst, zrows)

    out_full = pl.pallas_call(
        _tc3_body,
        grid=(grid,),
        in_specs=[
            pl.BlockSpec((BN, ROWW), lambda i: (i, 0)),
            pl.BlockSpec((BN, ROWW), lambda i: (i + grid, 0)),
            pl.BlockSpec((H, HC), lambda i: (0, 0)),
        ],
        out_specs=pl.BlockSpec((BN, HC), lambda i: (i, 0)),
        out_shape=jax.ShapeDtypeStruct((NPAD, HC), f32),
    )(acc, acc, p_exp)

    return out_full[:N]


# speedup vs baseline: 2.4481x; 1.0152x over previous
import jax
import jax.numpy as jnp
from jax import lax
from jax.experimental import pallas as pl
from jax.experimental.pallas import tpu as pltpu
from jax.experimental.pallas import tpu_sc as plsc

N = 10000
E = 320000
D = 128
H = 8
C = 16
HC = H * C
NPAD = 10240
ROWW = 144
NC = 2
NS = 16
NW = NC * NS
EPW = E // NW
CH = 128
NFULL = EPW // CH
TAIL = EPW - NFULL * CH
BN = 1024
RPT = NPAD // NS

_HIGH = lax.Precision.HIGHEST


def _tc12_body(x_ref, wq_ref, bq_ref, ab_ref, p_ref, e8_ref, t_ref,
               q_s, qa_s, gm_s):
    p = pl.program_id(0)
    i = pl.program_id(1)

    @pl.when(p == 0)
    def _():
        xq = jnp.dot(x_ref[...], wq_ref[...], precision=_HIGH) + bq_ref[...]
        q_s[pl.ds(i * BN, BN), :] = xq
        qa = jnp.dot(xq, ab_ref[...], precision=_HIGH)
        qa_s[pl.ds(i * BN, BN), :] = qa
        bm = jnp.max(qa, axis=0, keepdims=True)

        @pl.when(i == 0)
        def _():
            gm_s[...] = bm

        @pl.when(i != 0)
        def _():
            gm_s[...] = jnp.maximum(gm_s[...], bm)

    @pl.when(p == 1)
    def _():
        qz = jnp.exp(qa_s[pl.ds(i * BN, BN), :] - gm_s[...])
        qzrep = jnp.dot(qz, p_ref[...], precision=_HIGH)
        t_ref[:, :HC] = q_s[pl.ds(i * BN, BN), :] * qzrep
        t_ref[:, HC:ROWW] = jnp.dot(qz, e8_ref[...], precision=_HIGH)


def _tc3_body(a0_ref, a1_ref, p_ref, o_ref):
    s = a0_ref[...] + a1_ref[...]
    den = s[:, HC:HC + H]
    dexp = jnp.dot(den, p_ref[...], precision=_HIGH)
    o_ref[...] = jnp.maximum(s[:, :HC] / (dexp + 1e-16), 0.0)


def _sc_body(t_hbm, src_hbm, dst_hbm, z_hbm, outa_hbm, outb_hbm,
             sidx0, didx0, sidx1, didx1, rows0, rows1, didx_t, accum,
             sem0, sem1):
    c = lax.axis_index("c")
    s = lax.axis_index("s")
    base = pl.multiple_of((c * NS + s) * EPW, 8)
    rb = pl.multiple_of(s * RPT, 8)

    def ldidx(j, sidx, didx):
        eb = pl.multiple_of(base + j * CH, 8)
        pltpu.sync_copy(src_hbm.at[pl.ds(eb, CH)], sidx)
        pltpu.sync_copy(dst_hbm.at[pl.ds(eb, CH)], didx)

    def start(sidx, buf, sem):
        pltpu.async_copy(t_hbm.at[sidx], buf, sem)

    def wait(buf, sem):
        pltpu.make_async_copy(t_hbm.at[sidx0], buf, sem).wait()

    def scat(didx, buf):
        pltpu.sync_copy(buf, accum.at[didx], add=True)

    pltpu.sync_copy(z_hbm.at[pl.ds(rb, RPT)], accum.at[pl.ds(rb, RPT)])
    ldidx(0, sidx0, didx0)
    start(sidx0, rows0, sem0)
    plsc.subcore_barrier()

    def body(jj, carry):
        j = 2 * jj
        ldidx(j + 1, sidx1, didx1)
        start(sidx1, rows1, sem1)
        wait(rows0, sem0)
        scat(didx0, rows0)
        ldidx(j + 2, sidx0, didx0)
        start(sidx0, rows0, sem0)
        wait(rows1, sem1)
        scat(didx1, rows1)
        return carry

    lax.fori_loop(0, (NFULL - 2) // 2, body, 0, unroll=False)

    ldidx(NFULL - 1, sidx1, didx1)
    start(sidx1, rows1, sem1)
    wait(rows0, sem0)
    scat(didx0, rows0)
    wait(rows1, sem1)
    scat(didx1, rows1)

    eb = pl.multiple_of(base + NFULL * CH, 8)
    pltpu.sync_copy(src_hbm.at[pl.ds(eb, TAIL)], sidx0.at[pl.ds(0, TAIL)])
    pltpu.sync_copy(dst_hbm.at[pl.ds(eb, TAIL)], didx_t)
    pltpu.async_copy(t_hbm.at[sidx0.at[pl.ds(0, TAIL)]],
                     rows0.at[pl.ds(0, TAIL)], sem0).wait()
    pltpu.sync_copy(rows0.at[pl.ds(0, TAIL)], accum.at[didx_t], add=True)

    plsc.subcore_barrier()

    @pl.when(c == 0)
    def _():
        pltpu.sync_copy(accum.at[pl.ds(rb, RPT)], outa_hbm.at[pl.ds(rb, RPT)])

    @pl.when(c == 1)
    def _():
        pltpu.sync_copy(accum.at[pl.ds(rb, RPT)], outb_hbm.at[pl.ds(rb, RPT)])


def kernel(x, Wq, bq, Wk, bk, A, edge_index):
    del Wk, bk
    f32 = jnp.float32
    x_pad = jnp.pad(x, ((0, NPAD - N), (0, 0)))
    ab = (A.T[:, :, None] * jnp.eye(H, dtype=f32)[:, None, :]).reshape(HC, H)
    p_exp = jnp.kron(jnp.eye(H, dtype=f32), jnp.ones((1, C), f32))
    e8 = jnp.concatenate([jnp.eye(H, dtype=f32),
                          jnp.zeros((H, ROWW - HC - H), f32)], axis=1)
    bq2 = bq.reshape(1, HC)
    src = edge_index[0]
    dst = edge_index[1]
    zrows = jnp.zeros((NPAD, ROWW), f32)

    grid = NPAD // BN
    t_tab = pl.pallas_call(
        _tc12_body,
        grid=(2, grid),
        in_specs=[
            pl.BlockSpec((BN, D), lambda p, i: (i * (1 - p), 0)),
            pl.BlockSpec((D, HC), lambda p, i: (0, 0)),
            pl.BlockSpec((1, HC), lambda p, i: (0, 0)),
            pl.BlockSpec((HC, H), lambda p, i: (0, 0)),
            pl.BlockSpec((H, HC), lambda p, i: (0, 0)),
            pl.BlockSpec((H, ROWW - HC), lambda p, i: (0, 0)),
        ],
        out_specs=pl.BlockSpec((BN, ROWW),
                               lambda p, i: (grid * (1 - p) + i * p, 0)),
        out_shape=jax.ShapeDtypeStruct((NPAD + BN, ROWW), f32),
        scratch_shapes=[
            pltpu.VMEM((NPAD, HC), f32),
            pltpu.VMEM((NPAD, H), f32),
            pltpu.VMEM((1, H), f32),
        ],
    )(x_pad, Wq, bq2, ab, p_exp, e8)

    mesh = plsc.VectorSubcoreMesh(core_axis_name="c", subcore_axis_name="s",
                                  num_cores=NC, num_subcores=NS)
    acc0, acc1 = pl.kernel(
        _sc_body,
        out_type=[jax.ShapeDtypeStruct((NPAD, ROWW), f32),
                  jax.ShapeDtypeStruct((NPAD, ROWW), f32)],
        mesh=mesh,
        scratch_types=[
            pltpu.VMEM((CH,), jnp.int32),
            pltpu.VMEM((CH,), jnp.int32),
            pltpu.VMEM((CH,), jnp.int32),
            pltpu.VMEM((CH,), jnp.int32),
            pltpu.VMEM((CH, ROWW), f32),
            pltpu.VMEM((CH, ROWW), f32),
            pltpu.VMEM((TAIL,), jnp.int32),
            pltpu.VMEM_SHARED((NPAD, ROWW), f32),
            pltpu.SemaphoreType.DMA,
            pltpu.SemaphoreType.DMA,
        ],
        compiler_params=pltpu.CompilerParams(use_tc_tiling_on_sc=False),
    )(t_tab, src, dst, zrows)

    BN3 = 1000
    out = pl.pallas_call(
        _tc3_body,
        grid=(N // BN3,),
        in_specs=[
            pl.BlockSpec((BN3, ROWW), lambda i: (i, 0)),
            pl.BlockSpec((BN3, ROWW), lambda i: (i, 0)),
            pl.BlockSpec((H, HC), lambda i: (0, 0)),
        ],
        out_specs=pl.BlockSpec((BN3, HC), lambda i: (i, 0)),
        out_shape=jax.ShapeDtypeStruct((N, HC), f32),
    )(acc0, acc1, p_exp)

    return out


# speedup vs baseline: 2.6767x; 1.0934x over previous
import jax
import jax.numpy as jnp
from jax import lax
from jax.experimental import pallas as pl
from jax.experimental.pallas import tpu as pltpu
from jax.experimental.pallas import tpu_sc as plsc

N = 10000
E = 320000
D = 128
H = 8
C = 16
HC = H * C
NPAD = 10240
ROWW = 144
NC = 2
NS = 16
NW = NC * NS
EPW = E // NW
CH = 128
NFULL = EPW // CH
TAIL = EPW - NFULL * CH
BN = 1024
RPT = NPAD // NS

_HIGH = lax.Precision.HIGHEST


def _tc12_body(x_ref, wq_ref, bq_ref, ab_ref, p_ref, e8_ref, t_ref,
               q_s, qa_s, gm_s):
    p = pl.program_id(0)
    i = pl.program_id(1)

    @pl.when(p == 0)
    def _():
        xq = jnp.dot(x_ref[...], wq_ref[...], precision=_HIGH) + bq_ref[...]
        q_s[pl.ds(i * BN, BN), :] = xq
        qa = jnp.dot(xq, ab_ref[...], precision=_HIGH)
        qa_s[pl.ds(i * BN, BN), :] = qa
        bm = jnp.max(qa, axis=0, keepdims=True)

        @pl.when(i == 0)
        def _():
            gm_s[...] = bm

        @pl.when(i != 0)
        def _():
            gm_s[...] = jnp.maximum(gm_s[...], bm)

    @pl.when(p == 1)
    def _():
        qz = jnp.exp(qa_s[pl.ds(i * BN, BN), :] - gm_s[...])
        qzrep = jnp.dot(qz, p_ref[...], precision=_HIGH)
        t_ref[:, :HC] = q_s[pl.ds(i * BN, BN), :] * qzrep
        t_ref[:, HC:ROWW] = jnp.dot(qz, e8_ref[...], precision=_HIGH)


def _tc3_body(a0_ref, a1_ref, b0_ref, b1_ref, p_ref, o_ref):
    num = a0_ref[0] + a1_ref[0]
    den = b0_ref[0, :, :H] + b1_ref[0, :, :H]
    dexp = jnp.dot(den, p_ref[...], precision=_HIGH)
    o_ref[...] = jnp.maximum(num / (dexp + 1e-16), 0.0)


def _sc_body(t_hbm, ei_hbm, z_hbm, outa_hbm, outb_hbm,
             sidx0, didx0, sidx1, didx1, rows0, rows1, didx_t, accum,
             sem0, sem1):
    c = lax.axis_index("c")
    s = lax.axis_index("s")
    base = pl.multiple_of((c * NS + s) * EPW, 8)
    rb = pl.multiple_of(s * RPT, 8)

    def ldidx(j, sidx, didx):
        eb = pl.multiple_of(base + j * CH, 8)
        pltpu.sync_copy(ei_hbm.at[pl.ds(eb, CH)], sidx)
        pltpu.sync_copy(ei_hbm.at[pl.ds(E + eb, CH)], didx)

    def start(sidx, buf, sem):
        pltpu.async_copy(t_hbm.at[sidx], buf, sem)

    def wait(buf, sem):
        pltpu.make_async_copy(t_hbm.at[sidx0], buf, sem).wait()

    def scat(didx, buf):
        pltpu.sync_copy(buf, accum.at[didx], add=True)

    pltpu.sync_copy(z_hbm.at[pl.ds(rb, RPT)], accum.at[pl.ds(rb, RPT)])
    ldidx(0, sidx0, didx0)
    start(sidx0, rows0, sem0)
    plsc.subcore_barrier()

    def body(jj, carry):
        j = 2 * jj
        ldidx(j + 1, sidx1, didx1)
        start(sidx1, rows1, sem1)
        wait(rows0, sem0)
        scat(didx0, rows0)
        ldidx(j + 2, sidx0, didx0)
        start(sidx0, rows0, sem0)
        wait(rows1, sem1)
        scat(didx1, rows1)
        return carry

    lax.fori_loop(0, (NFULL - 2) // 2, body, 0, unroll=False)

    ldidx(NFULL - 1, sidx1, didx1)
    start(sidx1, rows1, sem1)
    wait(rows0, sem0)
    scat(didx0, rows0)
    wait(rows1, sem1)
    scat(didx1, rows1)

    eb = pl.multiple_of(base + NFULL * CH, 8)
    pltpu.sync_copy(ei_hbm.at[pl.ds(eb, TAIL)], sidx0.at[pl.ds(0, TAIL)])
    pltpu.sync_copy(ei_hbm.at[pl.ds(E + eb, TAIL)], didx_t)
    pltpu.async_copy(t_hbm.at[sidx0.at[pl.ds(0, TAIL)]],
                     rows0.at[pl.ds(0, TAIL)], sem0).wait()
    pltpu.sync_copy(rows0.at[pl.ds(0, TAIL)], accum.at[didx_t], add=True)

    plsc.subcore_barrier()

    pltpu.sync_copy(accum.at[pl.ds(rb, RPT), pl.ds(0, HC)],
                    outa_hbm.at[c, pl.ds(rb, RPT)])
    pltpu.sync_copy(accum.at[pl.ds(rb, RPT), pl.ds(HC, ROWW - HC)],
                    outb_hbm.at[c, pl.ds(rb, RPT)])


def kernel(x, Wq, bq, Wk, bk, A, edge_index):
    del Wk, bk
    f32 = jnp.float32
    x_pad = jnp.pad(x, ((0, NPAD - N), (0, 0)))
    ab = (A.T[:, :, None] * jnp.eye(H, dtype=f32)[:, None, :]).reshape(HC, H)
    p_exp = jnp.kron(jnp.eye(H, dtype=f32), jnp.ones((1, C), f32))
    e8 = jnp.concatenate([jnp.eye(H, dtype=f32),
                          jnp.zeros((H, ROWW - HC - H), f32)], axis=1)
    bq2 = bq.reshape(1, HC)
    eflat = edge_index.reshape(2 * E)
    zrows = jnp.zeros((NPAD, ROWW), f32)

    grid = NPAD // BN
    t_tab = pl.pallas_call(
        _tc12_body,
        grid=(2, grid),
        in_specs=[
            pl.BlockSpec((BN, D), lambda p, i: (i * (1 - p), 0)),
            pl.BlockSpec((D, HC), lambda p, i: (0, 0)),
            pl.BlockSpec((1, HC), lambda p, i: (0, 0)),
            pl.BlockSpec((HC, H), lambda p, i: (0, 0)),
            pl.BlockSpec((H, HC), lambda p, i: (0, 0)),
            pl.BlockSpec((H, ROWW - HC), lambda p, i: (0, 0)),
        ],
        out_specs=pl.BlockSpec((BN, ROWW),
                               lambda p, i: (grid * (1 - p) + i * p, 0)),
        out_shape=jax.ShapeDtypeStruct((NPAD + BN, ROWW), f32),
        scratch_shapes=[
            pltpu.VMEM((NPAD, HC), f32),
            pltpu.VMEM((NPAD, H), f32),
            pltpu.VMEM((1, H), f32),
        ],
    )(x_pad, Wq, bq2, ab, p_exp, e8)

    mesh = plsc.VectorSubcoreMesh(core_axis_name="c", subcore_axis_name="s",
                                  num_cores=NC, num_subcores=NS)
    acca, accb = pl.kernel(
        _sc_body,
        out_type=[jax.ShapeDtypeStruct((NC, NPAD, HC), f32),
                  jax.ShapeDtypeStruct((NC, NPAD, ROWW - HC), f32)],
        mesh=mesh,
        scratch_types=[
            pltpu.VMEM((CH,), jnp.int32),
            pltpu.VMEM((CH,), jnp.int32),
            pltpu.VMEM((CH,), jnp.int32),
            pltpu.VMEM((CH,), jnp.int32),
            pltpu.VMEM((CH, ROWW), f32),
            pltpu.VMEM((CH, ROWW), f32),
            pltpu.VMEM((TAIL,), jnp.int32),
            pltpu.VMEM_SHARED((NPAD, ROWW), f32),
            pltpu.SemaphoreType.DMA,
            pltpu.SemaphoreType.DMA,
        ],
        compiler_params=pltpu.CompilerParams(use_tc_tiling_on_sc=False),
    )(t_tab, eflat, zrows)

    BN3 = 1000
    out = pl.pallas_call(
        _tc3_body,
        grid=(N // BN3,),
        in_specs=[
            pl.BlockSpec((1, BN3, HC), lambda i: (0, i, 0)),
            pl.BlockSpec((1, BN3, HC), lambda i: (1, i, 0)),
            pl.BlockSpec((1, BN3, ROWW - HC), lambda i: (0, i, 0)),
            pl.BlockSpec((1, BN3, ROWW - HC), lambda i: (1, i, 0)),
            pl.BlockSpec((H, HC), lambda i: (0, 0)),
        ],
        out_specs=pl.BlockSpec((BN3, HC), lambda i: (i, 0)),
        out_shape=jax.ShapeDtypeStruct((N, HC), f32),
    )(acca, acca, accb, accb, p_exp)

    return out


# speedup vs baseline: 2.7904x; 1.0425x over previous
import jax
import jax.numpy as jnp
from jax import lax
from jax.experimental import pallas as pl
from jax.experimental.pallas import tpu as pltpu
from jax.experimental.pallas import tpu_sc as plsc

N = 10000
E = 320000
D = 128
H = 8
C = 16
HC = H * C
NPAD = 10240
ROWW = 144
NC = 2
NS = 16
NW = NC * NS
EPW = E // NW
CH = 128
NFULL = EPW // CH
TAIL = EPW - NFULL * CH
BN = 2000
NB = N // BN
RPT = NPAD // NS

_HIGH = lax.Precision.HIGHEST


def _tc12_body(x_ref, wq_ref, bq_ref, ab_ref, p_ref, e8_ref, t_ref,
               q_s, qa_s, gm_s):
    p = pl.program_id(0)
    i = pl.program_id(1)

    @pl.when(p == 0)
    def _():
        xq = jnp.dot(x_ref[...], wq_ref[...], precision=_HIGH) + bq_ref[...]
        q_s[pl.ds(i * BN, BN), :] = xq
        qa = jnp.dot(xq, ab_ref[...], precision=_HIGH)
        qa_s[pl.ds(i * BN, BN), :] = qa
        bm = jnp.max(qa, axis=0, keepdims=True)

        @pl.when(i == 0)
        def _():
            gm_s[...] = bm

        @pl.when(i != 0)
        def _():
            gm_s[...] = jnp.maximum(gm_s[...], bm)

    @pl.when(p == 1)
    def _():
        qz = jnp.exp(qa_s[pl.ds(i * BN, BN), :] - gm_s[...])
        qzrep = jnp.dot(qz, p_ref[...], precision=_HIGH)
        t_ref[:, :HC] = q_s[pl.ds(i * BN, BN), :] * qzrep
        t_ref[:, HC:ROWW] = jnp.dot(qz, e8_ref[...], precision=_HIGH)


def _tc3_body(a0_ref, a1_ref, b0_ref, b1_ref, p_ref, o_ref):
    num = a0_ref[0] + a1_ref[0]
    den = b0_ref[0, :, :H] + b1_ref[0, :, :H]
    dexp = jnp.dot(den, p_ref[...], precision=_HIGH)
    o_ref[...] = jnp.maximum(num / (dexp + 1e-16), 0.0)


def _sc_body(t_hbm, ei_hbm, z_hbm, outa_hbm, outb_hbm,
             sidx0, didx0, sidx1, didx1, rows0, rows1, didx_t, accum,
             sem0, sem1):
    c = lax.axis_index("c")
    s = lax.axis_index("s")
    base = pl.multiple_of((c * NS + s) * EPW, 8)
    rb = pl.multiple_of(s * RPT, 8)

    def ldidx(j, sidx, didx):
        eb = pl.multiple_of(base + j * CH, 8)
        pltpu.sync_copy(ei_hbm.at[pl.ds(eb, CH)], sidx)
        pltpu.sync_copy(ei_hbm.at[pl.ds(E + eb, CH)], didx)

    def start(sidx, buf, sem):
        pltpu.async_copy(t_hbm.at[sidx], buf, sem)

    def wait(buf, sem):
        pltpu.make_async_copy(t_hbm.at[sidx0], buf, sem).wait()

    def scat(didx, buf):
        pltpu.sync_copy(buf, accum.at[didx], add=True)

    pltpu.sync_copy(z_hbm.at[pl.ds(rb, RPT)], accum.at[pl.ds(rb, RPT)])
    ldidx(0, sidx0, didx0)
    start(sidx0, rows0, sem0)
    plsc.subcore_barrier()

    def body(jj, carry):
        j = 2 * jj
        ldidx(j + 1, sidx1, didx1)
        start(sidx1, rows1, sem1)
        wait(rows0, sem0)
        scat(didx0, rows0)
        ldidx(j + 2, sidx0, didx0)
        start(sidx0, rows0, sem0)
        wait(rows1, sem1)
        scat(didx1, rows1)
        return carry

    lax.fori_loop(0, (NFULL - 2) // 2, body, 0, unroll=False)

    ldidx(NFULL - 1, sidx1, didx1)
    start(sidx1, rows1, sem1)
    wait(rows0, sem0)
    scat(didx0, rows0)
    wait(rows1, sem1)
    scat(didx1, rows1)

    eb = pl.multiple_of(base + NFULL * CH, 8)
    pltpu.sync_copy(ei_hbm.at[pl.ds(eb, TAIL)], sidx0.at[pl.ds(0, TAIL)])
    pltpu.sync_copy(ei_hbm.at[pl.ds(E + eb, TAIL)], didx_t)
    pltpu.async_copy(t_hbm.at[sidx0.at[pl.ds(0, TAIL)]],
                     rows0.at[pl.ds(0, TAIL)], sem0).wait()
    pltpu.sync_copy(rows0.at[pl.ds(0, TAIL)], accum.at[didx_t], add=True)

    plsc.subcore_barrier()

    pltpu.sync_copy(accum.at[pl.ds(rb, RPT), pl.ds(0, HC)],
                    outa_hbm.at[c, pl.ds(rb, RPT)])
    pltpu.sync_copy(accum.at[pl.ds(rb, RPT), pl.ds(HC, ROWW - HC)],
                    outb_hbm.at[c, pl.ds(rb, RPT)])


def kernel(x, Wq, bq, Wk, bk, A, edge_index):
    del Wk, bk
    f32 = jnp.float32
    ab = (A.T[:, :, None] * jnp.eye(H, dtype=f32)[:, None, :]).reshape(HC, H)
    p_exp = jnp.kron(jnp.eye(H, dtype=f32), jnp.ones((1, C), f32))
    e8 = jnp.concatenate([jnp.eye(H, dtype=f32),
                          jnp.zeros((H, ROWW - HC - H), f32)], axis=1)
    bq2 = bq.reshape(1, HC)
    eflat = edge_index.reshape(2 * E)
    zrows = jnp.zeros((NPAD, ROWW), f32)

    t_tab = pl.pallas_call(
        _tc12_body,
        grid=(2, NB),
        in_specs=[
            pl.BlockSpec((BN, D), lambda p, i: (i * (1 - p), 0)),
            pl.BlockSpec((D, HC), lambda p, i: (0, 0)),
            pl.BlockSpec((1, HC), lambda p, i: (0, 0)),
            pl.BlockSpec((HC, H), lambda p, i: (0, 0)),
            pl.BlockSpec((H, HC), lambda p, i: (0, 0)),
            pl.BlockSpec((H, ROWW - HC), lambda p, i: (0, 0)),
        ],
        out_specs=pl.BlockSpec((BN, ROWW),
                               lambda p, i: (NB * (1 - p) + i * p, 0)),
        out_shape=jax.ShapeDtypeStruct((N + BN, ROWW), f32),
        scratch_shapes=[
            pltpu.VMEM((N, HC), f32),
            pltpu.VMEM((N, H), f32),
            pltpu.VMEM((1, H), f32),
        ],
    )(x, Wq, bq2, ab, p_exp, e8)

    mesh = plsc.VectorSubcoreMesh(core_axis_name="c", subcore_axis_name="s",
                                  num_cores=NC, num_subcores=NS)
    acca, accb = pl.kernel(
        _sc_body,
        out_type=[jax.ShapeDtypeStruct((NC, NPAD, HC), f32),
                  jax.ShapeDtypeStruct((NC, NPAD, ROWW - HC), f32)],
        mesh=mesh,
        scratch_types=[
            pltpu.VMEM((CH,), jnp.int32),
            pltpu.VMEM((CH,), jnp.int32),
            pltpu.VMEM((CH,), jnp.int32),
            pltpu.VMEM((CH,), jnp.int32),
            pltpu.VMEM((CH, ROWW), f32),
            pltpu.VMEM((CH, ROWW), f32),
            pltpu.VMEM((TAIL,), jnp.int32),
            pltpu.VMEM_SHARED((NPAD, ROWW), f32),
            pltpu.SemaphoreType.DMA,
            pltpu.SemaphoreType.DMA,
        ],
        compiler_params=pltpu.CompilerParams(use_tc_tiling_on_sc=False),
    )(t_tab, eflat, zrows)

    BN3 = 2000
    out = pl.pallas_call(
        _tc3_body,
        grid=(N // BN3,),
        in_specs=[
            pl.BlockSpec((1, BN3, HC), lambda i: (0, i, 0)),
            pl.BlockSpec((1, BN3, HC), lambda i: (1, i, 0)),
            pl.BlockSpec((1, BN3, ROWW - HC), lambda i: (0, i, 0)),
            pl.BlockSpec((1, BN3, ROWW - HC), lambda i: (1, i, 0)),
            pl.BlockSpec((H, HC), lambda i: (0, 0)),
        ],
        out_specs=pl.BlockSpec((BN3, HC), lambda i: (i, 0)),
        out_shape=jax.ShapeDtypeStruct((N, HC), f32),
    )(acca, acca, accb, accb, p_exp)

    return out
